# baseline (speedup 1.0000x reference)
import functools

import jax
import jax.numpy as jnp
from jax import lax
from jax.experimental import pallas as pl
from jax.experimental.pallas import tpu as pltpu
from jax.experimental.pallas import tpu_sc as plsc

_BN = 1000
_BE = 4000
_C = 80
_NC = 2
_NS = 16


def _sc_agg(h, ea, src, dst):
    n, hdim = h.shape
    e = src.shape[0]
    nw = _NC * _NS
    npad = ((n + 8 * nw - 1) // (8 * nw)) * (8 * nw)
    rpt = npad // nw
    cb = 2560
    nchunk = e // cb
    nv = cb // 16
    fb = 128
    fthr = fb - 16

    mesh = plsc.VectorSubcoreMesh(core_axis_name="c", subcore_axis_name="s")

    @functools.partial(
        pl.kernel,
        out_type=jax.ShapeDtypeStruct((npad, hdim), jnp.float32),
        mesh=mesh,
        scratch_types=[
            pltpu.VMEM((cb,), jnp.int32),
            pltpu.VMEM((cb,), jnp.int32),
            pltpu.VMEM((fb,), jnp.int32),
            pltpu.VMEM((fb,), jnp.int32),
            pltpu.VMEM((fb,), jnp.int32),
            pltpu.VMEM((fb, hdim), jnp.float32),
            pltpu.VMEM((fb, hdim), jnp.float32),
            pltpu.VMEM((rpt, hdim), jnp.float32),
            pltpu.SemaphoreType.DMA,
            pltpu.SemaphoreType.DMA,
        ],
    )
    def k(h_hbm, ea_hbm, src_hbm, dst_hbm, out_hbm,
          dstb, srcb, dbuf, sbuf, ebuf, hrows, earows, acc, sem_h, sem_e):
        cid = lax.axis_index("c")
        sid = lax.axis_index("s")
        wid = sid * _NC + cid
        lo = wid * rpt
        hi = lo + rpt
        zi = jnp.zeros((16,), jnp.int32)
        zf = jnp.zeros((16,), jnp.float32)

        def zrow(i, _):
            for j in range(hdim // 16):
                acc[i, pl.ds(j * 16, 16)] = zf
            return 0
        lax.fori_loop(0, rpt, zrow, 0)
        for j in range(fb // 16):
            sbuf[pl.ds(j * 16, 16)] = zi
            ebuf[pl.ds(j * 16, 16)] = zi

        lanes = lax.iota(jnp.int32, 16)

        def edge_body(e2, fl):
            g16 = (e2 // 16) * 16
            l = e2 - g16
            dvec = dbuf[pl.ds(g16, 16)]
            dloc = dvec[(lanes + l) & 15][0]
            for j in range(hdim // 16):
                sl = pl.ds(j * 16, 16)
                m = jnp.maximum(hrows[e2, sl] + earows[e2, sl], 0.0)
                acc[dloc, sl] = acc[dloc, sl] + m
            return fl

        def fire(fl):
            cph = pltpu.async_copy(h_hbm.at[sbuf], hrows, sem_h)
            cpe = pltpu.async_copy(ea_hbm.at[ebuf], earows, sem_e)
            cph.wait()
            cpe.wait()
            lax.fori_loop(0, fl, edge_body, fl)
            return jnp.int32(0)

        gv = 4
        none = gv * 16

        def chunk_body(c, fill):
            off = c * cb
            pltpu.sync_copy(dst_hbm.at[pl.ds(off, cb)], dstb)
            pltpu.sync_copy(src_hbm.at[pl.ds(off, cb)], srcb)

            def gloop(g, fill):
                gb = g * (16 * gv)
                encs = []
                cm = jnp.zeros((16,), jnp.int32)
                for q in range(gv):
                    dv = dstb[pl.ds(gb + q * 16, 16)]
                    mi = (jnp.where(dv >= lo, 1, 0)
                          + jnp.where(dv < hi, 1, 0))
                    encs.append(jnp.where(mi == 2, q * 16 + lanes, none))
                    cm = cm + jnp.where(mi == 2, 1, 0)
                for d in (1, 2, 4, 8):
                    cm = cm + cm[lanes ^ d]
                cnt = cm[0]

                def mbody(i, st):
                    e0, e1, e2_, e3, fill = st
                    comb = jnp.minimum(jnp.minimum(e0, e1),
                                       jnp.minimum(e2_, e3))
                    for d in (1, 2, 4, 8):
                        comb = jnp.minimum(comb, comb[lanes ^ d])
                    m0 = comb[0]
                    g16 = (m0 // 16) * 16
                    l0 = m0 - g16
                    fl16 = fill & 15
                    fbase = (fill // 16) * 16
                    dv = dstb[pl.ds(gb + g16, 16)]
                    sv = srcb[pl.ds(gb + g16, 16)]
                    rot = (lanes + (l0 - fl16)) & 15
                    ins = lanes == fl16
                    dbuf[pl.ds(fbase, 16)] = jnp.where(
                        ins, dv[rot] - lo, dbuf[pl.ds(fbase, 16)])
                    sbuf[pl.ds(fbase, 16)] = jnp.where(
                        ins, sv[rot], sbuf[pl.ds(fbase, 16)])
                    ebuf[pl.ds(fbase, 16)] = jnp.where(
                        ins, off + gb + m0, ebuf[pl.ds(fbase, 16)])
                    fill = fill + 1
                    qsel = m0 // 16
                    t0 = jnp.where(qsel == 0, l0, -1)
                    t1 = jnp.where(qsel == 1, l0, -1)
                    t2 = jnp.where(qsel == 2, l0, -1)
                    t3 = jnp.where(qsel == 3, l0, -1)
                    e0 = jnp.where(lanes == t0, none, e0)
                    e1 = jnp.where(lanes == t1, none, e1)
                    e2_ = jnp.where(lanes == t2, none, e2_)
                    e3 = jnp.where(lanes == t3, none, e3)
                    fill = lax.cond(fill > fthr, fire, lambda f: f, fill)
                    return (e0, e1, e2_, e3, fill)

                st = lax.fori_loop(
                    0, cnt, mbody,
                    (encs[0], encs[1], encs[2], encs[3], fill))
                return st[4]
            return lax.fori_loop(0, cb // (16 * gv), gloop, fill)

        fill = lax.fori_loop(0, nchunk, chunk_body, jnp.int32(0))
        lax.cond(fill > 0, fire, lambda f: f, fill)
        pltpu.sync_copy(acc, out_hbm.at[pl.ds(lo, rpt)])

    return k(h, ea, src, dst)[:n]


def _ea_body(ea_ref, w_ref, b_ref, out_ref):
    out_ref[0] = (jnp.dot(ea_ref[...], w_ref[0],
                          preferred_element_type=jnp.float32) + b_ref[0])


def _ea_all(edge_attr, we, be):
    e, ed = edge_attr.shape
    nl, _, hdim = we.shape
    nb = e // _BE
    return pl.pallas_call(
        _ea_body,
        grid=(nl, nb),
        in_specs=[
            pl.BlockSpec((_BE, ed), lambda l, j: (j, 0)),
            pl.BlockSpec((1, ed, hdim), lambda l, j: (l, 0, 0)),
            pl.BlockSpec((1, 1, hdim), lambda l, j: (l, 0, 0)),
        ],
        out_specs=pl.BlockSpec((1, _BE, hdim), lambda l, j: (l, j, 0)),
        out_shape=jax.ShapeDtypeStruct((nl, e, hdim), jnp.float32),
    )(edge_attr, we, be)


def _fold8(a):
    b = a[0:4] + a[4:8]
    c = b[0:2] + b[2:4]
    return c[0:1] + c[1:2]


def _mlp_body(h_ref, a_ref, w1_ref, b1_ref, w2_ref, b2_ref,
              u_ref, st_ref):
    i = pl.program_id(0)
    t = h_ref[...] + a_ref[...]
    u1 = jnp.maximum(jnp.dot(t, w1_ref[...],
                             preferred_element_type=jnp.float32) + b1_ref[...], 0.0)
    u = jnp.dot(u1, w2_ref[...], preferred_element_type=jnp.float32) + b2_ref[...]
    u_ref[...] = u

    @pl.when(i == 0)
    def _():
        st_ref[...] = jnp.zeros_like(st_ref)

    def body(k, acc):
        return acc + u_ref[pl.ds(8 * k, 8), :]
    st_ref[0] = lax.fori_loop(0, _BN // 8, body, st_ref[0])


def _mlp_stats(h, agg, w1, b1, w2, b2):
    n, hdim = h.shape
    nb = n // _BN
    return pl.pallas_call(
        _mlp_body,
        grid=(nb,),
        in_specs=[
            pl.BlockSpec((_BN, hdim), lambda i: (i, 0)),
            pl.BlockSpec((_BN, hdim), lambda i: (i, 0)),
            pl.BlockSpec((hdim, hdim), lambda i: (0, 0)),
            pl.BlockSpec((1, hdim), lambda i: (0, 0)),
            pl.BlockSpec((hdim, hdim), lambda i: (0, 0)),
            pl.BlockSpec((1, hdim), lambda i: (0, 0)),
        ],
        out_specs=[
            pl.BlockSpec((_BN, hdim), lambda i: (i, 0)),
            pl.BlockSpec((2, 8, hdim), lambda i: (0, 0, 0)),
        ],
        out_shape=[
            jax.ShapeDtypeStruct((n, hdim), jnp.float32),
            jax.ShapeDtypeStruct((2, 8, hdim), jnp.float32),
        ],
    )(h, agg, w1, b1, w2, b2)


def _mean_of(st_ref, rn):
    return _fold8(st_ref[0]) * rn


def _var_of(sq_ref, rn):
    return (_fold8(sq_ref[0]) + _fold8(sq_ref[1])) * rn


def _var_body(nbh, rn, u_ref, st_ref, sq_ref):
    i = pl.program_id(0)
    m = _mean_of(st_ref, rn)

    @pl.when(i == 0)
    def _():
        sq_ref[...] = jnp.zeros_like(sq_ref)

    def seq_accum(half):
        def body(k, acc):
            d = u_ref[pl.ds(8 * k, 8), :] - m
            return acc + d * d
        sq_ref[half] = lax.fori_loop(0, _BN // 8, body, sq_ref[half])

    @pl.when(i < nbh)
    def _():
        seq_accum(0)

    @pl.when(i >= nbh)
    def _():
        seq_accum(1)


def _var_stats(u, stats):
    n, hdim = u.shape
    nb = n // _BN
    return pl.pallas_call(
        functools.partial(_var_body, nb // 2, 1.0 / n),
        grid=(nb,),
        in_specs=[
            pl.BlockSpec((_BN, hdim), lambda i: (i, 0)),
            pl.BlockSpec((2, 8, hdim), lambda i: (0, 0, 0)),
        ],
        out_specs=pl.BlockSpec((2, 8, hdim), lambda i: (0, 0, 0)),
        out_shape=jax.ShapeDtypeStruct((2, 8, hdim), jnp.float32),
    )(u, stats)


def _bnr_body(rn, u_ref, st_ref, sq_ref, g_ref, b_ref, o_ref):
    m = _mean_of(st_ref, rn)
    v = _var_of(sq_ref, rn)
    o_ref[...] = jnp.maximum(
        g_ref[...] * (u_ref[...] - m) * (1.0 / jnp.sqrt(v + 1e-5))
        + b_ref[...], 0.0)


def _bn_relu(u, stats, sqstats, gamma, beta):
    n, hdim = u.shape
    nb = n // _BN
    return pl.pallas_call(
        functools.partial(_bnr_body, 1.0 / n),
        grid=(nb,),
        in_specs=[
            pl.BlockSpec((_BN, hdim), lambda i: (i, 0)),
            pl.BlockSpec((2, 8, hdim), lambda i: (0, 0, 0)),
            pl.BlockSpec((2, 8, hdim), lambda i: (0, 0, 0)),
            pl.BlockSpec((1, hdim), lambda i: (0, 0)),
            pl.BlockSpec((1, hdim), lambda i: (0, 0)),
        ],
        out_specs=pl.BlockSpec((_BN, hdim), lambda i: (i, 0)),
        out_shape=jax.ShapeDtypeStruct((n, hdim), jnp.float32),
    )(u, stats, sqstats, gamma, beta)


def _pool_body(g, h_ref, b_ref, z_ref, c_ref):
    i = pl.program_id(0)
    bidx = b_ref[0, 0, :]
    bn = bidx.shape[0]
    oh = (bidx[:, None] == lax.broadcasted_iota(jnp.int32, (bn, g), 1)
          ).astype(jnp.float32)
    zp = lax.dot_general(oh, h_ref[...], (((0,), (0,)), ((), ())),
                         preferred_element_type=jnp.float32,
                         precision=lax.Precision.HIGHEST)
    cp = lax.dot_general(oh, jnp.ones((bn, 8), jnp.float32),
                         (((0,), (0,)), ((), ())),
                         preferred_element_type=jnp.float32,
                         precision=lax.Precision.HIGHEST)

    @pl.when(i == 0)
    def _():
        z_ref[...] = zp
        c_ref[...] = cp

    @pl.when(i != 0)
    def _():
        z_ref[...] = z_ref[...] + zp
        c_ref[...] = c_ref[...] + cp


def _pool(h, batch, g):
    n, hdim = h.shape
    nb = n // _BN
    batch3 = batch.reshape(nb, 1, _BN)
    return pl.pallas_call(
        functools.partial(_pool_body, g),
        grid=(nb,),
        in_specs=[
            pl.BlockSpec((_BN, hdim), lambda i: (i, 0)),
            pl.BlockSpec((1, 1, _BN), lambda i: (i, 0, 0)),
        ],
        out_specs=[
            pl.BlockSpec((g, hdim), lambda i: (0, 0)),
            pl.BlockSpec((g, 8), lambda i: (0, 0)),
        ],
        out_shape=[
            jax.ShapeDtypeStruct((g, hdim), jnp.float32),
            jax.ShapeDtypeStruct((g, 8), jnp.float32),
        ],
    )(h, batch3)


def _dec_body(h_ref, w1_ref, b1_ref, w2_ref, b2_ref, o_ref):
    u = jnp.maximum(jnp.dot(h_ref[...], w1_ref[...],
                            preferred_element_type=jnp.float32) + b1_ref[...],
                    0.0)
    o_ref[...] = jnp.dot(u, w2_ref[...],
                         preferred_element_type=jnp.float32) + b2_ref[...]


def _decode(h, w1, b1, w2, b2):
    n, hdim = h.shape
    d = w2.shape[1]
    nb = n // _BN
    return pl.pallas_call(
        _dec_body,
        grid=(nb,),
        in_specs=[
            pl.BlockSpec((_BN, hdim), lambda i: (i, 0)),
            pl.BlockSpec((hdim, hdim), lambda i: (0, 0)),
            pl.BlockSpec((1, hdim), lambda i: (0, 0)),
            pl.BlockSpec((hdim, d), lambda i: (0, 0)),
            pl.BlockSpec((1, d), lambda i: (0, 0)),
        ],
        out_specs=pl.BlockSpec((_BN, d), lambda i: (i, 0)),
        out_shape=jax.ShapeDtypeStruct((n, d), jnp.float32),
    )(h, w1, b1, w2, b2)


def _bn_rows(r, g_row, b_row):
    rn = 1.0 / r.shape[0]
    acc = r[0:8]
    for k in range(1, r.shape[0] // 8):
        acc = acc + r[8 * k:8 * k + 8]
    m = _fold8(acc) * rn
    d = r - m
    sq = d * d
    acc2 = sq[0:8]
    for k in range(1, r.shape[0] // 8):
        acc2 = acc2 + sq[8 * k:8 * k + 8]
    v = _fold8(acc2) * rn
    return g_row * d * (1.0 / jnp.sqrt(v + 1e-5)) + b_row


def _head_body(zs_ref, c_ref, nf_ref,
               w0a_ref, w0b_ref, b0_ref, g0_ref, be0_ref,
               w1_ref, b1_ref, g1_ref, be1_ref,
               s1w_ref, s1b_ref, s2w_ref, s2b_ref,
               h1w_ref, h1b_ref, h1g_ref, h1be_ref,
               h2w_ref, h2b_ref,
               y_ref, z_ref):
    cnt = c_ref[...][:, :1]
    z = zs_ref[...] / jnp.maximum(cnt, 1.0)
    z_ref[...] = z
    r = (jnp.dot(z, w0a_ref[...], preferred_element_type=jnp.float32)
         + jnp.dot(nf_ref[...], w0b_ref[...], preferred_element_type=jnp.float32)
         + b0_ref[...])
    r = jnp.maximum(_bn_rows(r, g0_ref[...], be0_ref[...]), 0.0)
    r = jnp.dot(r, w1_ref[...], preferred_element_type=jnp.float32) + b1_ref[...]
    r = jnp.maximum(_bn_rows(r, g1_ref[...], be1_ref[...]), 0.0)
    s = jnp.maximum(jnp.dot(r, s1w_ref[...],
                            preferred_element_type=jnp.float32) + s1b_ref[...],
                    0.0)
    s = jnp.dot(s, s2w_ref[...], preferred_element_type=jnp.float32) + s2b_ref[...]
    w = 1.0 / (1.0 + jnp.exp(-s))
    r = r * w
    hh = jnp.dot(r, h1w_ref[...], preferred_element_type=jnp.float32) + h1b_ref[...]
    hh = jnp.maximum(_bn_rows(hh, h1g_ref[...], h1be_ref[...]), 0.0)
    y_ref[...] = (jnp.dot(hh, h2w_ref[...], preferred_element_type=jnp.float32)
                  + h2b_ref[...])


def _head(zsum, cnt8, nf8, p):
    g, hdim = zsum.shape
    w0 = p["blocks"][0]["W"]
    w0a = w0[:hdim]
    w0b = jnp.pad(w0[hdim:], ((0, 8 - (w0.shape[0] - hdim)), (0, 0)))
    gr = w0.shape[1]
    sr = p["se1"]["W"].shape[1]
    hh = p["head1"]["W"].shape[1]
    args = (
        zsum, cnt8, nf8,
        w0a, w0b, p["blocks"][0]["b"][None],
        p["blocks"][0]["gamma"][None], p["blocks"][0]["beta"][None],
        p["blocks"][1]["W"], p["blocks"][1]["b"][None],
        p["blocks"][1]["gamma"][None], p["blocks"][1]["beta"][None],
        p["se1"]["W"], p["se1"]["b"][None],
        p["se2"]["W"], p["se2"]["b"][None],
        p["head1"]["W"], p["head1"]["b"][None],
        p["head1"]["gamma"][None], p["head1"]["beta"][None],
        p["head2"]["W"], p["head2"]["b"][None],
    )
    return pl.pallas_call(
        _head_body,
        in_specs=[pl.BlockSpec(a.shape, lambda: (0,) * a.ndim) for a in args],
        out_specs=[
            pl.BlockSpec((g, 1), lambda: (0, 0)),
            pl.BlockSpec((g, hdim), lambda: (0, 0)),
        ],
        out_shape=[
            jax.ShapeDtypeStruct((g, 1), jnp.float32),
            jax.ShapeDtypeStruct((g, hdim), jnp.float32),
        ],
    )(*args)


def kernel(x, edge_index, edge_attr, batch, num_feats, params):
    n, hdim = x.shape
    src = edge_index[0]
    dst = edge_index[1]
    conv = params["conv"]
    g = num_feats.shape[0]

    we = jnp.stack([p["edge"]["W"] for p in conv])
    be = jnp.stack([p["edge"]["b"] for p in conv])[:, None, :]
    eas = _ea_all(edge_attr, we, be)

    h = x
    for l, p in enumerate(conv):
        agg = _sc_agg(h, eas[l], src, dst)
        u, stats = _mlp_stats(h, agg,
                              p["mlp1"]["W"], p["mlp1"]["b"][None],
                              p["mlp2"]["W"], p["mlp2"]["b"][None])
        sqstats = _var_stats(u, stats)
        h = _bn_relu(u, stats, sqstats, p["gamma"][None], p["beta"][None])

    zsum, cnt8 = _pool(h, batch, g)
    nf8 = jnp.pad(num_feats, ((0, 0), (0, 8 - num_feats.shape[1])))
    y_hat, z128 = _head(zsum, cnt8, nf8, params)
    z = jnp.concatenate([z128, num_feats], axis=1)
    x_logits = _decode(h, params["dec1"]["W"], params["dec1"]["b"][None],
                       params["dec2"]["W"], params["dec2"]["b"][None])
    return (y_hat, x_logits, h, z)


# speedup vs baseline: 1.0117x; 1.0117x over previous
import functools

import jax
import jax.numpy as jnp
from jax import lax
from jax.experimental import pallas as pl
from jax.experimental.pallas import tpu as pltpu
from jax.experimental.pallas import tpu_sc as plsc

_BN = 1000
_BE = 4000
_C = 80
_NC = 2
_NS = 16


def _sc_agg(h, ea, src, dst):
    n, hdim = h.shape
    e = src.shape[0]
    nw = _NC * _NS
    npad = ((n + 8 * nw - 1) // (8 * nw)) * (8 * nw)
    rpt = npad // nw
    cb = 6400
    nchunk = e // cb
    nv = cb // 16
    fb = 128
    fthr = fb - 16

    mesh = plsc.VectorSubcoreMesh(core_axis_name="c", subcore_axis_name="s")

    @functools.partial(
        pl.kernel,
        out_type=jax.ShapeDtypeStruct((npad, hdim), jnp.float32),
        mesh=mesh,
        scratch_types=[
            pltpu.VMEM((cb,), jnp.int32),
            pltpu.VMEM((cb,), jnp.int32),
            pltpu.VMEM((fb,), jnp.int32),
            pltpu.VMEM((fb,), jnp.int32),
            pltpu.VMEM((fb,), jnp.int32),
            pltpu.VMEM((fb, hdim), jnp.float32),
            pltpu.VMEM((fb, hdim), jnp.float32),
            pltpu.VMEM((rpt, hdim), jnp.float32),
            pltpu.SemaphoreType.DMA,
            pltpu.SemaphoreType.DMA,
            pltpu.SemaphoreType.DMA,
            pltpu.SemaphoreType.DMA,
        ],
    )
    def k(h_hbm, ea_hbm, src_hbm, dst_hbm, out_hbm,
          dstb, srcb, dbuf, sbuf, ebuf, hrows, earows, acc, sem_h, sem_e,
          sem_d, sem_s):
        cid = lax.axis_index("c")
        sid = lax.axis_index("s")
        wid = sid * _NC + cid
        lo = wid * rpt
        hi = lo + rpt
        zi = jnp.zeros((16,), jnp.int32)
        zf = jnp.zeros((16,), jnp.float32)

        def zrow(i, _):
            for j in range(hdim // 16):
                acc[i, pl.ds(j * 16, 16)] = zf
            return 0
        lax.fori_loop(0, rpt, zrow, 0)
        for j in range(fb // 16):
            sbuf[pl.ds(j * 16, 16)] = zi
            ebuf[pl.ds(j * 16, 16)] = zi

        lanes = lax.iota(jnp.int32, 16)

        def edge_body(e2, fl):
            g16 = (e2 // 16) * 16
            l = e2 - g16
            dvec = dbuf[pl.ds(g16, 16)]
            dloc = dvec[(lanes + l) & 15][0]
            for j in range(hdim // 16):
                sl = pl.ds(j * 16, 16)
                m = jnp.maximum(hrows[e2, sl] + earows[e2, sl], 0.0)
                acc[dloc, sl] = acc[dloc, sl] + m
            return fl

        def fire(fl):
            cph = pltpu.async_copy(h_hbm.at[sbuf], hrows, sem_h)
            cpe = pltpu.async_copy(ea_hbm.at[ebuf], earows, sem_e)
            cph.wait()
            cpe.wait()
            lax.fori_loop(0, fl, edge_body, fl)
            return jnp.int32(0)

        gv = 8
        none = gv * 16

        def chunk_body(c, fill):
            off = c * cb
            cpd = pltpu.async_copy(dst_hbm.at[pl.ds(off, cb)], dstb, sem_d)
            cps = pltpu.async_copy(src_hbm.at[pl.ds(off, cb)], srcb, sem_s)
            cpd.wait()
            cps.wait()

            def gloop(g, fill):
                gb = g * (16 * gv)
                encs = []
                cm = jnp.zeros((16,), jnp.int32)
                for q in range(gv):
                    dv = dstb[pl.ds(gb + q * 16, 16)]
                    mi = (jnp.where(dv >= lo, 1, 0)
                          + jnp.where(dv < hi, 1, 0))
                    encs.append(jnp.where(mi == 2, q * 16 + lanes, none))
                    cm = cm + jnp.where(mi == 2, 1, 0)
                for d in (1, 2, 4, 8):
                    cm = cm + cm[lanes ^ d]
                cnt = cm[0]

                def mbody(i, st):
                    fill = st[gv]
                    es = list(st[:gv])
                    comb = es[0]
                    for q in range(1, gv):
                        comb = jnp.minimum(comb, es[q])
                    for d in (1, 2, 4, 8):
                        comb = jnp.minimum(comb, comb[lanes ^ d])
                    m0 = comb[0]
                    g16 = (m0 // 16) * 16
                    l0 = m0 - g16
                    fl16 = fill & 15
                    fbase = (fill // 16) * 16
                    dv = dstb[pl.ds(gb + g16, 16)]
                    sv = srcb[pl.ds(gb + g16, 16)]
                    rot = (lanes + (l0 - fl16)) & 15
                    ins = lanes == fl16
                    dbuf[pl.ds(fbase, 16)] = jnp.where(
                        ins, dv[rot] - lo, dbuf[pl.ds(fbase, 16)])
                    sbuf[pl.ds(fbase, 16)] = jnp.where(
                        ins, sv[rot], sbuf[pl.ds(fbase, 16)])
                    ebuf[pl.ds(fbase, 16)] = jnp.where(
                        ins, off + gb + m0, ebuf[pl.ds(fbase, 16)])
                    fill = fill + 1
                    qsel = m0 // 16
                    for q in range(gv):
                        tq = jnp.where(qsel == q, l0, -1)
                        es[q] = jnp.where(lanes == tq, none, es[q])
                    fill = lax.cond(fill > fthr, fire, lambda f: f, fill)
                    return tuple(es) + (fill,)

                st = lax.fori_loop(0, cnt, mbody, tuple(encs) + (fill,))
                return st[gv]
            return lax.fori_loop(0, cb // (16 * gv), gloop, fill)

        fill = lax.fori_loop(0, nchunk, chunk_body, jnp.int32(0))
        lax.cond(fill > 0, fire, lambda f: f, fill)
        pltpu.sync_copy(acc, out_hbm.at[pl.ds(lo, rpt)])

    return k(h, ea, src, dst)[:n]


def _ea_body(ea_ref, w_ref, b_ref, out_ref):
    out_ref[0] = (jnp.dot(ea_ref[...], w_ref[0],
                          preferred_element_type=jnp.float32) + b_ref[0])


def _ea_all(edge_attr, we, be):
    e, ed = edge_attr.shape
    nl, _, hdim = we.shape
    nb = e // _BE
    return pl.pallas_call(
        _ea_body,
        grid=(nl, nb),
        in_specs=[
            pl.BlockSpec((_BE, ed), lambda l, j: (j, 0)),
            pl.BlockSpec((1, ed, hdim), lambda l, j: (l, 0, 0)),
            pl.BlockSpec((1, 1, hdim), lambda l, j: (l, 0, 0)),
        ],
        out_specs=pl.BlockSpec((1, _BE, hdim), lambda l, j: (l, j, 0)),
        out_shape=jax.ShapeDtypeStruct((nl, e, hdim), jnp.float32),
    )(edge_attr, we, be)


def _fold8(a):
    b = a[0:4] + a[4:8]
    c = b[0:2] + b[2:4]
    return c[0:1] + c[1:2]


def _mlp_body(h_ref, a_ref, w1_ref, b1_ref, w2_ref, b2_ref,
              u_ref, st_ref):
    i = pl.program_id(0)
    t = h_ref[...] + a_ref[...]
    u1 = jnp.maximum(jnp.dot(t, w1_ref[...],
                             preferred_element_type=jnp.float32) + b1_ref[...], 0.0)
    u = jnp.dot(u1, w2_ref[...], preferred_element_type=jnp.float32) + b2_ref[...]
    u_ref[...] = u

    @pl.when(i == 0)
    def _():
        st_ref[...] = jnp.zeros_like(st_ref)

    def body(k, acc):
        return acc + u_ref[pl.ds(8 * k, 8), :]
    st_ref[0] = lax.fori_loop(0, _BN // 8, body, st_ref[0])


def _mlp_stats(h, agg, w1, b1, w2, b2):
    n, hdim = h.shape
    nb = n // _BN
    return pl.pallas_call(
        _mlp_body,
        grid=(nb,),
        in_specs=[
            pl.BlockSpec((_BN, hdim), lambda i: (i, 0)),
            pl.BlockSpec((_BN, hdim), lambda i: (i, 0)),
            pl.BlockSpec((hdim, hdim), lambda i: (0, 0)),
            pl.BlockSpec((1, hdim), lambda i: (0, 0)),
            pl.BlockSpec((hdim, hdim), lambda i: (0, 0)),
            pl.BlockSpec((1, hdim), lambda i: (0, 0)),
        ],
        out_specs=[
            pl.BlockSpec((_BN, hdim), lambda i: (i, 0)),
            pl.BlockSpec((2, 8, hdim), lambda i: (0, 0, 0)),
        ],
        out_shape=[
            jax.ShapeDtypeStruct((n, hdim), jnp.float32),
            jax.ShapeDtypeStruct((2, 8, hdim), jnp.float32),
        ],
    )(h, agg, w1, b1, w2, b2)


def _mean_of(st_ref, rn):
    return _fold8(st_ref[0]) * rn


def _var_of(sq_ref, rn):
    return (_fold8(sq_ref[0]) + _fold8(sq_ref[1])) * rn


def _var_body(nbh, rn, u_ref, st_ref, sq_ref):
    i = pl.program_id(0)
    m = _mean_of(st_ref, rn)

    @pl.when(i == 0)
    def _():
        sq_ref[...] = jnp.zeros_like(sq_ref)

    def seq_accum(half):
        def body(k, acc):
            d = u_ref[pl.ds(8 * k, 8), :] - m
            return acc + d * d
        sq_ref[half] = lax.fori_loop(0, _BN // 8, body, sq_ref[half])

    @pl.when(i < nbh)
    def _():
        seq_accum(0)

    @pl.when(i >= nbh)
    def _():
        seq_accum(1)


def _var_stats(u, stats):
    n, hdim = u.shape
    nb = n // _BN
    return pl.pallas_call(
        functools.partial(_var_body, nb // 2, 1.0 / n),
        grid=(nb,),
        in_specs=[
            pl.BlockSpec((_BN, hdim), lambda i: (i, 0)),
            pl.BlockSpec((2, 8, hdim), lambda i: (0, 0, 0)),
        ],
        out_specs=pl.BlockSpec((2, 8, hdim), lambda i: (0, 0, 0)),
        out_shape=jax.ShapeDtypeStruct((2, 8, hdim), jnp.float32),
    )(u, stats)


def _bnr_body(rn, u_ref, st_ref, sq_ref, g_ref, b_ref, o_ref):
    m = _mean_of(st_ref, rn)
    v = _var_of(sq_ref, rn)
    o_ref[...] = jnp.maximum(
        g_ref[...] * (u_ref[...] - m) * (1.0 / jnp.sqrt(v + 1e-5))
        + b_ref[...], 0.0)


def _bn_relu(u, stats, sqstats, gamma, beta):
    n, hdim = u.shape
    nb = n // _BN
    return pl.pallas_call(
        functools.partial(_bnr_body, 1.0 / n),
        grid=(nb,),
        in_specs=[
            pl.BlockSpec((_BN, hdim), lambda i: (i, 0)),
            pl.BlockSpec((2, 8, hdim), lambda i: (0, 0, 0)),
            pl.BlockSpec((2, 8, hdim), lambda i: (0, 0, 0)),
            pl.BlockSpec((1, hdim), lambda i: (0, 0)),
            pl.BlockSpec((1, hdim), lambda i: (0, 0)),
        ],
        out_specs=pl.BlockSpec((_BN, hdim), lambda i: (i, 0)),
        out_shape=jax.ShapeDtypeStruct((n, hdim), jnp.float32),
    )(u, stats, sqstats, gamma, beta)


def _pool_body(g, h_ref, b_ref, z_ref, c_ref):
    i = pl.program_id(0)
    bidx = b_ref[0, 0, :]
    bn = bidx.shape[0]
    oh = (bidx[:, None] == lax.broadcasted_iota(jnp.int32, (bn, g), 1)
          ).astype(jnp.float32)
    zp = lax.dot_general(oh, h_ref[...], (((0,), (0,)), ((), ())),
                         preferred_element_type=jnp.float32,
                         precision=lax.Precision.HIGHEST)
    cp = lax.dot_general(oh, jnp.ones((bn, 8), jnp.float32),
                         (((0,), (0,)), ((), ())),
                         preferred_element_type=jnp.float32,
                         precision=lax.Precision.HIGHEST)

    @pl.when(i == 0)
    def _():
        z_ref[...] = zp
        c_ref[...] = cp

    @pl.when(i != 0)
    def _():
        z_ref[...] = z_ref[...] + zp
        c_ref[...] = c_ref[...] + cp


def _pool(h, batch, g):
    n, hdim = h.shape
    nb = n // _BN
    batch3 = batch.reshape(nb, 1, _BN)
    return pl.pallas_call(
        functools.partial(_pool_body, g),
        grid=(nb,),
        in_specs=[
            pl.BlockSpec((_BN, hdim), lambda i: (i, 0)),
            pl.BlockSpec((1, 1, _BN), lambda i: (i, 0, 0)),
        ],
        out_specs=[
            pl.BlockSpec((g, hdim), lambda i: (0, 0)),
            pl.BlockSpec((g, 8), lambda i: (0, 0)),
        ],
        out_shape=[
            jax.ShapeDtypeStruct((g, hdim), jnp.float32),
            jax.ShapeDtypeStruct((g, 8), jnp.float32),
        ],
    )(h, batch3)


def _dec_body(h_ref, w1_ref, b1_ref, w2_ref, b2_ref, o_ref):
    u = jnp.maximum(jnp.dot(h_ref[...], w1_ref[...],
                            preferred_element_type=jnp.float32) + b1_ref[...],
                    0.0)
    o_ref[...] = jnp.dot(u, w2_ref[...],
                         preferred_element_type=jnp.float32) + b2_ref[...]


def _decode(h, w1, b1, w2, b2):
    n, hdim = h.shape
    d = w2.shape[1]
    nb = n // _BN
    return pl.pallas_call(
        _dec_body,
        grid=(nb,),
        in_specs=[
            pl.BlockSpec((_BN, hdim), lambda i: (i, 0)),
            pl.BlockSpec((hdim, hdim), lambda i: (0, 0)),
            pl.BlockSpec((1, hdim), lambda i: (0, 0)),
            pl.BlockSpec((hdim, d), lambda i: (0, 0)),
            pl.BlockSpec((1, d), lambda i: (0, 0)),
        ],
        out_specs=pl.BlockSpec((_BN, d), lambda i: (i, 0)),
        out_shape=jax.ShapeDtypeStruct((n, d), jnp.float32),
    )(h, w1, b1, w2, b2)


def _bn_rows(r, g_row, b_row):
    rn = 1.0 / r.shape[0]
    acc = r[0:8]
    for k in range(1, r.shape[0] // 8):
        acc = acc + r[8 * k:8 * k + 8]
    m = _fold8(acc) * rn
    d = r - m
    sq = d * d
    acc2 = sq[0:8]
    for k in range(1, r.shape[0] // 8):
        acc2 = acc2 + sq[8 * k:8 * k + 8]
    v = _fold8(acc2) * rn
    return g_row * d * (1.0 / jnp.sqrt(v + 1e-5)) + b_row


def _head_body(zs_ref, c_ref, nf_ref,
               w0a_ref, w0b_ref, b0_ref, g0_ref, be0_ref,
               w1_ref, b1_ref, g1_ref, be1_ref,
               s1w_ref, s1b_ref, s2w_ref, s2b_ref,
               h1w_ref, h1b_ref, h1g_ref, h1be_ref,
               h2w_ref, h2b_ref,
               y_ref, z_ref):
    cnt = c_ref[...][:, :1]
    z = zs_ref[...] / jnp.maximum(cnt, 1.0)
    z_ref[...] = z
    r = (jnp.dot(z, w0a_ref[...], preferred_element_type=jnp.float32)
         + jnp.dot(nf_ref[...], w0b_ref[...], preferred_element_type=jnp.float32)
         + b0_ref[...])
    r = jnp.maximum(_bn_rows(r, g0_ref[...], be0_ref[...]), 0.0)
    r = jnp.dot(r, w1_ref[...], preferred_element_type=jnp.float32) + b1_ref[...]
    r = jnp.maximum(_bn_rows(r, g1_ref[...], be1_ref[...]), 0.0)
    s = jnp.maximum(jnp.dot(r, s1w_ref[...],
                            preferred_element_type=jnp.float32) + s1b_ref[...],
                    0.0)
    s = jnp.dot(s, s2w_ref[...], preferred_element_type=jnp.float32) + s2b_ref[...]
    w = 1.0 / (1.0 + jnp.exp(-s))
    r = r * w
    hh = jnp.dot(r, h1w_ref[...], preferred_element_type=jnp.float32) + h1b_ref[...]
    hh = jnp.maximum(_bn_rows(hh, h1g_ref[...], h1be_ref[...]), 0.0)
    y_ref[...] = (jnp.dot(hh, h2w_ref[...], preferred_element_type=jnp.float32)
                  + h2b_ref[...])


def _head(zsum, cnt8, nf8, p):
    g, hdim = zsum.shape
    w0 = p["blocks"][0]["W"]
    w0a = w0[:hdim]
    w0b = jnp.pad(w0[hdim:], ((0, 8 - (w0.shape[0] - hdim)), (0, 0)))
    gr = w0.shape[1]
    sr = p["se1"]["W"].shape[1]
    hh = p["head1"]["W"].shape[1]
    args = (
        zsum, cnt8, nf8,
        w0a, w0b, p["blocks"][0]["b"][None],
        p["blocks"][0]["gamma"][None], p["blocks"][0]["beta"][None],
        p["blocks"][1]["W"], p["blocks"][1]["b"][None],
        p["blocks"][1]["gamma"][None], p["blocks"][1]["beta"][None],
        p["se1"]["W"], p["se1"]["b"][None],
        p["se2"]["W"], p["se2"]["b"][None],
        p["head1"]["W"], p["head1"]["b"][None],
        p["head1"]["gamma"][None], p["head1"]["beta"][None],
        p["head2"]["W"], p["head2"]["b"][None],
    )
    return pl.pallas_call(
        _head_body,
        in_specs=[pl.BlockSpec(a.shape, lambda: (0,) * a.ndim) for a in args],
        out_specs=[
            pl.BlockSpec((g, 1), lambda: (0, 0)),
            pl.BlockSpec((g, hdim), lambda: (0, 0)),
        ],
        out_shape=[
            jax.ShapeDtypeStruct((g, 1), jnp.float32),
            jax.ShapeDtypeStruct((g, hdim), jnp.float32),
        ],
    )(*args)


def kernel(x, edge_index, edge_attr, batch, num_feats, params):
    n, hdim = x.shape
    src = edge_index[0]
    dst = edge_index[1]
    conv = params["conv"]
    g = num_feats.shape[0]

    we = jnp.stack([p["edge"]["W"] for p in conv])
    be = jnp.stack([p["edge"]["b"] for p in conv])[:, None, :]
    eas = _ea_all(edge_attr, we, be)

    h = x
    for l, p in enumerate(conv):
        agg = _sc_agg(h, eas[l], src, dst)
        u, stats = _mlp_stats(h, agg,
                              p["mlp1"]["W"], p["mlp1"]["b"][None],
                              p["mlp2"]["W"], p["mlp2"]["b"][None])
        sqstats = _var_stats(u, stats)
        h = _bn_relu(u, stats, sqstats, p["gamma"][None], p["beta"][None])

    zsum, cnt8 = _pool(h, batch, g)
    nf8 = jnp.pad(num_feats, ((0, 0), (0, 8 - num_feats.shape[1])))
    y_hat, z128 = _head(zsum, cnt8, nf8, params)
    z = jnp.concatenate([z128, num_feats], axis=1)
    x_logits = _decode(h, params["dec1"]["W"], params["dec1"]["b"][None],
                       params["dec2"]["W"], params["dec2"]["b"][None])
    return (y_hat, x_logits, h, z)
